# baseline (speedup 1.0000x reference)
import jax
import jax.numpy as jnp
from jax.experimental import pallas as pl
from jax.experimental.pallas import tpu as pltpu


def _copy_kernel(src_ref, dst_ref, sem):
    copy = pltpu.make_async_copy(src_ref, dst_ref, sem)
    copy.start()
    copy.wait()


def kernel(prototypes):
    return pl.pallas_call(
        _copy_kernel,
        out_shape=jax.ShapeDtypeStruct(prototypes.shape, prototypes.dtype),
        in_specs=[pl.BlockSpec(memory_space=pl.ANY)],
        out_specs=pl.BlockSpec(memory_space=pl.ANY),
        scratch_shapes=[pltpu.SemaphoreType.DMA],
    )(prototypes)


# speedup vs baseline: 1.0041x; 1.0041x over previous
import jax
import jax.numpy as jnp
from jax.experimental import pallas as pl
from jax.experimental.pallas import tpu as pltpu


_NUM_CHUNKS = 16


def _copy_kernel(src_ref, dst_ref, sems):
    rows = src_ref.shape[0]
    chunk = rows // _NUM_CHUNKS
    copies = []
    for i in range(_NUM_CHUNKS):
        c = pltpu.make_async_copy(
            src_ref.at[pl.ds(i * chunk, chunk)],
            dst_ref.at[pl.ds(i * chunk, chunk)],
            sems.at[i],
        )
        c.start()
        copies.append(c)
    for c in copies:
        c.wait()


def kernel(prototypes):
    return pl.pallas_call(
        _copy_kernel,
        out_shape=jax.ShapeDtypeStruct(prototypes.shape, prototypes.dtype),
        in_specs=[pl.BlockSpec(memory_space=pl.ANY)],
        out_specs=pl.BlockSpec(memory_space=pl.ANY),
        scratch_shapes=[pltpu.SemaphoreType.DMA((_NUM_CHUNKS,))],
    )(prototypes)


# speedup vs baseline: 28.3223x; 28.2075x over previous
import jax
import jax.numpy as jnp
from jax.experimental import pallas as pl
from jax.experimental.pallas import tpu as pltpu


_BLOCK_ROWS = 1024


def _copy_kernel(src_ref, dst_ref):
    dst_ref[...] = src_ref[...]


def kernel(prototypes):
    rows = prototypes.shape[0]
    return pl.pallas_call(
        _copy_kernel,
        out_shape=jax.ShapeDtypeStruct(prototypes.shape, prototypes.dtype),
        grid=(rows // _BLOCK_ROWS,),
        in_specs=[pl.BlockSpec((_BLOCK_ROWS, prototypes.shape[1]), lambda i: (i, 0))],
        out_specs=pl.BlockSpec((_BLOCK_ROWS, prototypes.shape[1]), lambda i: (i, 0)),
    )(prototypes)


# speedup vs baseline: 34.6932x; 1.2249x over previous
import jax
import jax.numpy as jnp
from jax.experimental import pallas as pl
from jax.experimental.pallas import tpu as pltpu


_BLOCK_ROWS = 2048


def _copy_kernel(src_ref, dst_ref):
    dst_ref[...] = src_ref[...]


def kernel(prototypes):
    rows = prototypes.shape[0]
    return pl.pallas_call(
        _copy_kernel,
        out_shape=jax.ShapeDtypeStruct(prototypes.shape, prototypes.dtype),
        grid=(rows // _BLOCK_ROWS,),
        in_specs=[pl.BlockSpec((_BLOCK_ROWS, prototypes.shape[1]), lambda i: (i, 0))],
        out_specs=pl.BlockSpec((_BLOCK_ROWS, prototypes.shape[1]), lambda i: (i, 0)),
    )(prototypes)


# speedup vs baseline: 42.7766x; 1.2330x over previous
import jax
import jax.numpy as jnp
from jax.experimental import pallas as pl
from jax.experimental.pallas import tpu as pltpu


_BLOCK_ROWS = 4096


def _copy_kernel(src_ref, dst_ref):
    dst_ref[...] = src_ref[...]


def kernel(prototypes):
    rows = prototypes.shape[0]
    return pl.pallas_call(
        _copy_kernel,
        out_shape=jax.ShapeDtypeStruct(prototypes.shape, prototypes.dtype),
        grid=(rows // _BLOCK_ROWS,),
        in_specs=[pl.BlockSpec((_BLOCK_ROWS, prototypes.shape[1]), lambda i: (i, 0))],
        out_specs=pl.BlockSpec((_BLOCK_ROWS, prototypes.shape[1]), lambda i: (i, 0)),
    )(prototypes)
